# Initial kernel scaffold; baseline (speedup 1.0000x reference)
import jax
import jax.numpy as jnp
from jax.experimental import pallas as pl


def kernel(x, edge_index, edge_attr, batch, poles, zeros, params):
    raise NotImplementedError("write your pallas kernel here")


# speedup vs baseline: 1.1095x; 1.1095x over previous
import jax
import jax.numpy as jnp
from jax.experimental import pallas as pl
from functools import partial

_N = 50000
_E = 800000
_B = 128
_H = 64
_BE = 4000
_BN = 2000


def _relu(v):
    return jnp.maximum(v, 0.0)


def _dot(a, b):
    return jax.lax.dot_general(a, b, (((1,), (0,)), ((), ())),
                               preferred_element_type=jnp.float32)


def _proj_body(x_ref, w_ref, b_ref, o_ref):
    o_ref[...] = _relu(_dot(x_ref[...], w_ref[...]) + b_ref[...])


def _proj(x, W, b):
    n = x.shape[0]
    return pl.pallas_call(
        _proj_body,
        grid=(n // _BN,),
        in_specs=[
            pl.BlockSpec((_BN, x.shape[1]), lambda i: (i, 0)),
            pl.BlockSpec(W.shape, lambda i: (0, 0)),
            pl.BlockSpec((1, W.shape[1]), lambda i: (0, 0)),
        ],
        out_specs=pl.BlockSpec((_BN, W.shape[1]), lambda i: (i, 0)),
        out_shape=jax.ShapeDtypeStruct((n, W.shape[1]), jnp.float32),
    )(x, W, b.reshape(1, -1))


def _edge_body(attr_ref, hs_ref, we_ref, be_ref, wm1_ref, wm2_ref, bm_ref,
               m_ref):
    e = _relu(_dot(attr_ref[...], we_ref[...]) + be_ref[...])
    m_ref[...] = _relu(_dot(hs_ref[...], wm1_ref[...]) +
                       _dot(e, wm2_ref[...]) + bm_ref[...])


def _edge_stage(attr, hs, We, be, Wm, bm):
    Wm1, Wm2 = Wm[:_H], Wm[_H:]
    return pl.pallas_call(
        _edge_body,
        grid=(_E // _BE,),
        in_specs=[
            pl.BlockSpec((_BE, 7), lambda i: (i, 0)),
            pl.BlockSpec((_BE, _H), lambda i: (i, 0)),
            pl.BlockSpec((7, _H), lambda i: (0, 0)),
            pl.BlockSpec((1, _H), lambda i: (0, 0)),
            pl.BlockSpec((_H, _H), lambda i: (0, 0)),
            pl.BlockSpec((_H, _H), lambda i: (0, 0)),
            pl.BlockSpec((1, _H), lambda i: (0, 0)),
        ],
        out_specs=pl.BlockSpec((_BE, _H), lambda i: (i, 0)),
        out_shape=jax.ShapeDtypeStruct((_E, _H), jnp.float32),
    )(attr, hs, We, be.reshape(1, -1), Wm1, Wm2, bm.reshape(1, -1))


def _upd_body(h_ref, a_ref, w1_ref, w2_ref, b_ref, o_ref):
    o_ref[...] = _relu(_dot(h_ref[...], w1_ref[...]) +
                       _dot(a_ref[...], w2_ref[...]) + b_ref[...])


def _upd_stage(h, agg, Wu, bu):
    W1, W2 = Wu[:_H], Wu[_H:]
    return pl.pallas_call(
        _upd_body,
        grid=(_N // _BN,),
        in_specs=[
            pl.BlockSpec((_BN, _H), lambda i: (i, 0)),
            pl.BlockSpec((_BN, _H), lambda i: (i, 0)),
            pl.BlockSpec((_H, _H), lambda i: (0, 0)),
            pl.BlockSpec((_H, _H), lambda i: (0, 0)),
            pl.BlockSpec((1, _H), lambda i: (0, 0)),
        ],
        out_specs=pl.BlockSpec((_BN, _H), lambda i: (i, 0)),
        out_shape=jax.ShapeDtypeStruct((_N, _H), jnp.float32),
    )(h, agg, W1, W2, bu.reshape(1, -1))


def _pool_body(h_ref, b_ref, acc_ref):
    i = pl.program_id(0)

    @pl.when(i == 0)
    def _():
        acc_ref[...] = jnp.zeros_like(acc_ref)

    cols = jax.lax.broadcasted_iota(jnp.int32, (_BN, _B), 1)
    onehot = (b_ref[...] == cols).astype(jnp.float32)
    hh = jnp.concatenate(
        [h_ref[...], jnp.ones((_BN, 1), jnp.float32)], axis=1)
    acc_ref[...] += jax.lax.dot_general(
        onehot, hh, (((0,), (0,)), ((), ())),
        preferred_element_type=jnp.float32)


def _pool(h, batch_col):
    return pl.pallas_call(
        _pool_body,
        grid=(_N // _BN,),
        in_specs=[
            pl.BlockSpec((_BN, _H), lambda i: (i, 0)),
            pl.BlockSpec((_BN, 1), lambda i: (i, 0)),
        ],
        out_specs=pl.BlockSpec((_B, _H + 1), lambda i: (0, 0)),
        out_shape=jax.ShapeDtypeStruct((_B, _H + 1), jnp.float32),
    )(h, batch_col)


def _vals_body(attr_ref, meta_ref, wg1_ref, wg2_ref, wv1_ref, wv2_ref,
               wo1_ref, wo2_ref, bvec_ref, hot_ref, idx_ref):
    i = pl.program_id(0)

    @pl.when(i == 0)
    def _():
        hot_ref[...] = jnp.zeros_like(hot_ref)
        idx_ref[...] = jnp.full_like(idx_ref, -1.0)

    st = meta_ref[:, 0:1]
    dt = meta_ref[:, 1:2]
    eb = meta_ref[:, 2:3]
    lo = jnp.minimum(st, dt)
    hi = jnp.maximum(st, dt)
    m_gv = (lo == 0) & (hi == 2)
    m_vv = (lo == 1) & (hi == 2)
    m_ot = jnp.logical_not(m_gv | m_vv)

    cols = jax.lax.broadcasted_iota(jnp.int32, (_BE, _B), 1)
    seg = eb == cols

    attr = attr_ref[...]
    bo1 = bvec_ref[4:5, 0:32]
    bo2 = bvec_ref[5:6, 0:16]
    enc_ot = _dot(_relu(_dot(attr, wo1_ref[...]) + bo1), wo2_ref[...]) + bo2
    oh_ot = jnp.where(seg & m_ot, 1.0, 0.0)
    hot_ref[...] += jax.lax.dot_general(
        oh_ot, enc_ot, (((0,), (0,)), ((), ())),
        preferred_element_type=jnp.float32)

    ids = (jax.lax.broadcasted_iota(jnp.int32, (_BE, _B), 0).astype(jnp.float32)
           + (i * _BE).astype(jnp.float32))
    cand_gv = jnp.where(seg & m_gv, ids, -1.0)
    cand_vv = jnp.where(seg & m_vv, ids, -1.0)
    mx_gv = jnp.max(cand_gv, axis=0)[None, :]
    mx_vv = jnp.max(cand_vv, axis=0)[None, :]
    idx_ref[0:1, :] = jnp.maximum(idx_ref[0:1, :], mx_gv)
    idx_ref[1:2, :] = jnp.maximum(idx_ref[1:2, :], mx_vv)


def _vals_stage(attr, meta, p):
    bvec = jnp.zeros((6, 32), jnp.float32)
    bvec = bvec.at[0, :32].set(p['edge_gv']['b1'])
    bvec = bvec.at[1, :16].set(p['edge_gv']['b2'])
    bvec = bvec.at[2, :32].set(p['edge_vv']['b1'])
    bvec = bvec.at[3, :16].set(p['edge_vv']['b2'])
    bvec = bvec.at[4, :32].set(p['edge_other']['b1'])
    bvec = bvec.at[5, :16].set(p['edge_other']['b2'])
    full = lambda s: pl.BlockSpec(s, lambda i: (0, 0))
    return pl.pallas_call(
        _vals_body,
        grid=(_E // _BE,),
        in_specs=[
            pl.BlockSpec((_BE, 7), lambda i: (i, 0)),
            pl.BlockSpec((_BE, 3), lambda i: (i, 0)),
            full((7, 32)), full((32, 16)),
            full((7, 32)), full((32, 16)),
            full((7, 32)), full((32, 16)),
            full((6, 32)),
        ],
        out_specs=[
            pl.BlockSpec((_B, 16), lambda i: (0, 0)),
            pl.BlockSpec((2, _B), lambda i: (0, 0)),
        ],
        out_shape=[
            jax.ShapeDtypeStruct((_B, 16), jnp.float32),
            jax.ShapeDtypeStruct((2, _B), jnp.float32),
        ],
    )(attr, meta,
      p['edge_gv']['W1'], p['edge_gv']['W2'],
      p['edge_vv']['W1'], p['edge_vv']['W2'],
      p['edge_other']['W1'], p['edge_other']['W2'], bvec)


def _mlp2_(v, W1, b1, W2, b2):
    return _relu(v @ W1 + b1) @ W2 + b2


@jax.jit
def _run(x, edge_index, edge_attr, batch, poles, zeros, params):
    src = edge_index[0]
    dst = edge_index[1]

    h = _proj(x, params['W_in'], params['b_in'])
    for lp in params['gnn']:
        hs = jnp.take(h, src, axis=0)
        m = _edge_stage(edge_attr, hs, lp['W_edge'], lp['b_edge'],
                        lp['W_msg'], lp['b_msg'])
        agg = jax.ops.segment_sum(m, dst, num_segments=_N)
        h = _upd_stage(h, agg, lp['W_upd'], lp['b_upd'])

    acc = _pool(h, batch.reshape(-1, 1))
    counts = acc[:, _H]
    mean = acc[:, :_H] / jnp.clip(counts, 1.0)[:, None]
    mx = jax.ops.segment_max(h, batch, num_segments=_B)
    mx = jnp.where(jnp.isfinite(mx), mx, 0.0)
    t = params['topo']
    ht = _relu(jnp.concatenate([mean, mx], axis=-1) @ t['W1'] + t['b1'])
    ht = _relu(ht @ t['W2'] + t['b2'])
    mu_topo = ht @ t['Wmu'] + t['bmu']
    lv_topo = ht @ t['Wlv'] + t['blv']

    node_types = jnp.argmax(x, axis=-1).astype(jnp.int32)
    st = jnp.take(node_types, src)
    dt = jnp.take(node_types, dst)
    eb = jnp.take(batch, src)
    meta = jnp.stack([st, dt, eb], axis=1)
    h_ot, idxf = _vals_stage(edge_attr, meta, params)
    idx = idxf.astype(jnp.int32)
    safe = jnp.clip(idx, 0)
    rows_gv = jnp.take(edge_attr, safe[0], axis=0)
    rows_vv = jnp.take(edge_attr, safe[1], axis=0)
    eg = params['edge_gv']
    ev = params['edge_vv']
    h_gv = jnp.where((idx[0] >= 0)[:, None],
                     _mlp2_(rows_gv, eg['W1'], eg['b1'], eg['W2'], eg['b2']),
                     0.0)
    h_vv = jnp.where((idx[1] >= 0)[:, None],
                     _mlp2_(rows_vv, ev['W1'], ev['b1'], ev['W2'], ev['b2']),
                     0.0)
    v = params['values']
    hv = _relu(jnp.concatenate([h_gv, h_vv, h_ot], axis=-1) @ v['Wc']
               + v['bc'])
    mu_v = hv @ v['Wmu'] + v['bmu']
    lv_v = hv @ v['Wlv'] + v['blv']

    pp = params['pz_poles']
    hp = _relu(poles @ pp['W1'] + pp['b1']).sum(axis=1) @ pp['W2'] + pp['b2']
    pz_ = params['pz_zeros']
    hz = _relu(zeros @ pz_['W1'] + pz_['b1']).sum(axis=1) @ pz_['W2'] + pz_['b2']
    pc = params['pz']
    hpz = _relu(jnp.concatenate([hp, hz], axis=-1) @ pc['Wc'] + pc['bc'])
    mu_pz = hpz @ pc['Wmu'] + pc['bmu']
    lv_pz = hpz @ pc['Wlv'] + pc['blv']

    mu = jnp.concatenate([mu_topo, mu_v, mu_pz], axis=-1)
    logvar = jnp.concatenate([lv_topo, lv_v, lv_pz], axis=-1)
    eps = jax.random.normal(jax.random.key(1), mu.shape, jnp.float32)
    z = mu + eps * jnp.exp(0.5 * logvar)
    return z, mu, logvar


def kernel(x, edge_index, edge_attr, batch, poles, zeros, params):
    return _run(x, edge_index, edge_attr, batch, poles, zeros, params)


# speedup vs baseline: 1.1140x; 1.0040x over previous
import jax
import jax.numpy as jnp
from jax.experimental import pallas as pl
from functools import partial

_N = 50000
_E = 800000
_B = 128
_H = 64
_BE = 8000
_BN = 5000


def _relu(v):
    return jnp.maximum(v, 0.0)


def _dot(a, b):
    return jax.lax.dot_general(a, b, (((1,), (0,)), ((), ())),
                               preferred_element_type=jnp.float32)


def _proj_body(x_ref, w_ref, b_ref, o_ref):
    o_ref[...] = _relu(_dot(x_ref[...], w_ref[...]) + b_ref[...])


def _proj(x, W, b):
    n = x.shape[0]
    return pl.pallas_call(
        _proj_body,
        grid=(n // _BN,),
        in_specs=[
            pl.BlockSpec((_BN, x.shape[1]), lambda i: (i, 0)),
            pl.BlockSpec(W.shape, lambda i: (0, 0)),
            pl.BlockSpec((1, W.shape[1]), lambda i: (0, 0)),
        ],
        out_specs=pl.BlockSpec((_BN, W.shape[1]), lambda i: (i, 0)),
        out_shape=jax.ShapeDtypeStruct((n, W.shape[1]), jnp.float32),
    )(x, W, b.reshape(1, -1))


def _edge_body(attr_ref, hs_ref, we_ref, be_ref, wm1_ref, wm2_ref, bm_ref,
               m_ref):
    e = _relu(_dot(attr_ref[...], we_ref[...]) + be_ref[...])
    m_ref[...] = _relu(_dot(hs_ref[...], wm1_ref[...]) +
                       _dot(e, wm2_ref[...]) + bm_ref[...])


def _edge_stage(attr, hs, We, be, Wm, bm):
    Wm1, Wm2 = Wm[:_H], Wm[_H:]
    return pl.pallas_call(
        _edge_body,
        grid=(_E // _BE,),
        in_specs=[
            pl.BlockSpec((_BE, 7), lambda i: (i, 0)),
            pl.BlockSpec((_BE, _H), lambda i: (i, 0)),
            pl.BlockSpec((7, _H), lambda i: (0, 0)),
            pl.BlockSpec((1, _H), lambda i: (0, 0)),
            pl.BlockSpec((_H, _H), lambda i: (0, 0)),
            pl.BlockSpec((_H, _H), lambda i: (0, 0)),
            pl.BlockSpec((1, _H), lambda i: (0, 0)),
        ],
        out_specs=pl.BlockSpec((_BE, _H), lambda i: (i, 0)),
        out_shape=jax.ShapeDtypeStruct((_E, _H), jnp.float32),
    )(attr, hs, We, be.reshape(1, -1), Wm1, Wm2, bm.reshape(1, -1))


def _upd_body(h_ref, a_ref, w1_ref, w2_ref, b_ref, o_ref):
    o_ref[...] = _relu(_dot(h_ref[...], w1_ref[...]) +
                       _dot(a_ref[...], w2_ref[...]) + b_ref[...])


def _upd_stage(h, agg, Wu, bu):
    W1, W2 = Wu[:_H], Wu[_H:]
    return pl.pallas_call(
        _upd_body,
        grid=(_N // _BN,),
        in_specs=[
            pl.BlockSpec((_BN, _H), lambda i: (i, 0)),
            pl.BlockSpec((_BN, _H), lambda i: (i, 0)),
            pl.BlockSpec((_H, _H), lambda i: (0, 0)),
            pl.BlockSpec((_H, _H), lambda i: (0, 0)),
            pl.BlockSpec((1, _H), lambda i: (0, 0)),
        ],
        out_specs=pl.BlockSpec((_BN, _H), lambda i: (i, 0)),
        out_shape=jax.ShapeDtypeStruct((_N, _H), jnp.float32),
    )(h, agg, W1, W2, bu.reshape(1, -1))


def _pool_body(h_ref, b_ref, acc_ref):
    i = pl.program_id(0)

    @pl.when(i == 0)
    def _():
        acc_ref[...] = jnp.zeros_like(acc_ref)

    cols = jax.lax.broadcasted_iota(jnp.int32, (_BN, _B), 1)
    onehot = (b_ref[...] == cols).astype(jnp.float32)
    hh = jnp.concatenate(
        [h_ref[...], jnp.ones((_BN, 1), jnp.float32)], axis=1)
    acc_ref[...] += jax.lax.dot_general(
        onehot, hh, (((0,), (0,)), ((), ())),
        preferred_element_type=jnp.float32)


def _pool(h, batch_col):
    return pl.pallas_call(
        _pool_body,
        grid=(_N // _BN,),
        in_specs=[
            pl.BlockSpec((_BN, _H), lambda i: (i, 0)),
            pl.BlockSpec((_BN, 1), lambda i: (i, 0)),
        ],
        out_specs=pl.BlockSpec((_B, _H + 1), lambda i: (0, 0)),
        out_shape=jax.ShapeDtypeStruct((_B, _H + 1), jnp.float32),
    )(h, batch_col)


def _vals_body(attr_ref, meta_ref, wg1_ref, wg2_ref, wv1_ref, wv2_ref,
               wo1_ref, wo2_ref, bvec_ref, hot_ref, idx_ref):
    i = pl.program_id(0)

    @pl.when(i == 0)
    def _():
        hot_ref[...] = jnp.zeros_like(hot_ref)
        idx_ref[...] = jnp.full_like(idx_ref, -1.0)

    st = meta_ref[:, 0:1]
    dt = meta_ref[:, 1:2]
    eb = meta_ref[:, 2:3]
    lo = jnp.minimum(st, dt)
    hi = jnp.maximum(st, dt)
    m_gv = (lo == 0) & (hi == 2)
    m_vv = (lo == 1) & (hi == 2)
    m_ot = jnp.logical_not(m_gv | m_vv)

    cols = jax.lax.broadcasted_iota(jnp.int32, (_BE, _B), 1)
    seg = eb == cols

    attr = attr_ref[...]
    bo1 = bvec_ref[4:5, 0:32]
    bo2 = bvec_ref[5:6, 0:16]
    enc_ot = _dot(_relu(_dot(attr, wo1_ref[...]) + bo1), wo2_ref[...]) + bo2
    oh_ot = jnp.where(seg & m_ot, 1.0, 0.0)
    hot_ref[...] += jax.lax.dot_general(
        oh_ot, enc_ot, (((0,), (0,)), ((), ())),
        preferred_element_type=jnp.float32)

    ids = (jax.lax.broadcasted_iota(jnp.int32, (_BE, _B), 0).astype(jnp.float32)
           + (i * _BE).astype(jnp.float32))
    cand_gv = jnp.where(seg & m_gv, ids, -1.0)
    cand_vv = jnp.where(seg & m_vv, ids, -1.0)
    mx_gv = jnp.max(cand_gv, axis=0)[None, :]
    mx_vv = jnp.max(cand_vv, axis=0)[None, :]
    idx_ref[0:1, :] = jnp.maximum(idx_ref[0:1, :], mx_gv)
    idx_ref[1:2, :] = jnp.maximum(idx_ref[1:2, :], mx_vv)


def _vals_stage(attr, meta, p):
    bvec = jnp.zeros((6, 32), jnp.float32)
    bvec = bvec.at[0, :32].set(p['edge_gv']['b1'])
    bvec = bvec.at[1, :16].set(p['edge_gv']['b2'])
    bvec = bvec.at[2, :32].set(p['edge_vv']['b1'])
    bvec = bvec.at[3, :16].set(p['edge_vv']['b2'])
    bvec = bvec.at[4, :32].set(p['edge_other']['b1'])
    bvec = bvec.at[5, :16].set(p['edge_other']['b2'])
    full = lambda s: pl.BlockSpec(s, lambda i: (0, 0))
    return pl.pallas_call(
        _vals_body,
        grid=(_E // _BE,),
        in_specs=[
            pl.BlockSpec((_BE, 7), lambda i: (i, 0)),
            pl.BlockSpec((_BE, 3), lambda i: (i, 0)),
            full((7, 32)), full((32, 16)),
            full((7, 32)), full((32, 16)),
            full((7, 32)), full((32, 16)),
            full((6, 32)),
        ],
        out_specs=[
            pl.BlockSpec((_B, 16), lambda i: (0, 0)),
            pl.BlockSpec((2, _B), lambda i: (0, 0)),
        ],
        out_shape=[
            jax.ShapeDtypeStruct((_B, 16), jnp.float32),
            jax.ShapeDtypeStruct((2, _B), jnp.float32),
        ],
    )(attr, meta,
      p['edge_gv']['W1'], p['edge_gv']['W2'],
      p['edge_vv']['W1'], p['edge_vv']['W2'],
      p['edge_other']['W1'], p['edge_other']['W2'], bvec)


def _mlp2_(v, W1, b1, W2, b2):
    return _relu(v @ W1 + b1) @ W2 + b2


@jax.jit
def _run(x, edge_index, edge_attr, batch, poles, zeros, params):
    src = edge_index[0]
    dst = edge_index[1]

    h = _proj(x, params['W_in'], params['b_in'])
    for lp in params['gnn']:
        hs = jnp.take(h, src, axis=0)
        m = _edge_stage(edge_attr, hs, lp['W_edge'], lp['b_edge'],
                        lp['W_msg'], lp['b_msg'])
        agg = jax.ops.segment_sum(m, dst, num_segments=_N)
        h = _upd_stage(h, agg, lp['W_upd'], lp['b_upd'])

    acc = _pool(h, batch.reshape(-1, 1))
    counts = acc[:, _H]
    mean = acc[:, :_H] / jnp.clip(counts, 1.0)[:, None]
    mx = jax.ops.segment_max(h, batch, num_segments=_B)
    mx = jnp.where(jnp.isfinite(mx), mx, 0.0)
    t = params['topo']
    ht = _relu(jnp.concatenate([mean, mx], axis=-1) @ t['W1'] + t['b1'])
    ht = _relu(ht @ t['W2'] + t['b2'])
    mu_topo = ht @ t['Wmu'] + t['bmu']
    lv_topo = ht @ t['Wlv'] + t['blv']

    node_types = jnp.argmax(x, axis=-1).astype(jnp.int32)
    st = jnp.take(node_types, src)
    dt = jnp.take(node_types, dst)
    eb = jnp.take(batch, src)
    meta = jnp.stack([st, dt, eb], axis=1)
    h_ot, idxf = _vals_stage(edge_attr, meta, params)
    idx = idxf.astype(jnp.int32)
    safe = jnp.clip(idx, 0)
    rows_gv = jnp.take(edge_attr, safe[0], axis=0)
    rows_vv = jnp.take(edge_attr, safe[1], axis=0)
    eg = params['edge_gv']
    ev = params['edge_vv']
    h_gv = jnp.where((idx[0] >= 0)[:, None],
                     _mlp2_(rows_gv, eg['W1'], eg['b1'], eg['W2'], eg['b2']),
                     0.0)
    h_vv = jnp.where((idx[1] >= 0)[:, None],
                     _mlp2_(rows_vv, ev['W1'], ev['b1'], ev['W2'], ev['b2']),
                     0.0)
    v = params['values']
    hv = _relu(jnp.concatenate([h_gv, h_vv, h_ot], axis=-1) @ v['Wc']
               + v['bc'])
    mu_v = hv @ v['Wmu'] + v['bmu']
    lv_v = hv @ v['Wlv'] + v['blv']

    pp = params['pz_poles']
    hp = _relu(poles @ pp['W1'] + pp['b1']).sum(axis=1) @ pp['W2'] + pp['b2']
    pz_ = params['pz_zeros']
    hz = _relu(zeros @ pz_['W1'] + pz_['b1']).sum(axis=1) @ pz_['W2'] + pz_['b2']
    pc = params['pz']
    hpz = _relu(jnp.concatenate([hp, hz], axis=-1) @ pc['Wc'] + pc['bc'])
    mu_pz = hpz @ pc['Wmu'] + pc['bmu']
    lv_pz = hpz @ pc['Wlv'] + pc['blv']

    mu = jnp.concatenate([mu_topo, mu_v, mu_pz], axis=-1)
    logvar = jnp.concatenate([lv_topo, lv_v, lv_pz], axis=-1)
    eps = jax.random.normal(jax.random.key(1), mu.shape, jnp.float32)
    z = mu + eps * jnp.exp(0.5 * logvar)
    return z, mu, logvar


def kernel(x, edge_index, edge_attr, batch, poles, zeros, params):
    return _run(x, edge_index, edge_attr, batch, poles, zeros, params)
